# baseline (speedup 1.0000x reference)
import functools
import math

import jax
import jax.numpy as jnp
from jax import lax
from jax.experimental import pallas as pl
from jax.experimental.pallas import tpu as pltpu
from jax.experimental.pallas import tpu_sc as plsc

B, N, E, H, NL = 64, 600, 9600, 128, 4
NP = 640
EPS = 1e-5
_INV = 1.0 / math.sqrt(1.0 + EPS)

_NWORKERS = 32
_ROWS_PER_W = NP // _NWORKERS
_LANES = 16


def _f32(x):
    return x.astype(jnp.float32)


@functools.lru_cache(maxsize=None)
def _get_sc_build_adj():
    mesh = plsc.VectorSubcoreMesh(core_axis_name="c", subcore_axis_name="s")

    @functools.partial(
        pl.kernel,
        out_type=jax.ShapeDtypeStruct((NP * NP,), jnp.float32),
        mesh=mesh,
        compiler_params=pltpu.CompilerParams(needs_layout_passes=False),
        scratch_types=[
            pltpu.VMEM((E,), jnp.int32),
            pltpu.VMEM((E,), jnp.int32),
            pltpu.VMEM((_ROWS_PER_W * NP,), jnp.float32),
        ],
    )
    def _sc_build_adj(edges_hbm, a_hbm, src_v, dst_v, loc_v):
        wid = lax.axis_index("s") * 2 + lax.axis_index("c")
        lo = wid * _ROWS_PER_W
        base = lo * NP

        pltpu.sync_copy(edges_hbm.at[0], src_v)
        pltpu.sync_copy(edges_hbm.at[1], dst_v)

        def _zero(i, _):
            loc_v[pl.ds(i * _LANES, _LANES)] = jnp.zeros((_LANES,),
                                                         jnp.float32)
            return 0
        lax.fori_loop(0, _ROWS_PER_W * NP // _LANES, _zero, 0, unroll=8)

        ones = jnp.ones((_LANES,), jnp.float32)

        def _scan(i, _):
            d16 = dst_v[pl.ds(i * _LANES, _LANES)]
            s16 = src_v[pl.ds(i * _LANES, _LANES)]
            inr = (d16 >= lo) & (d16 < lo + _ROWS_PER_W)
            idx = (d16 - lo) * NP + s16
            plsc.addupdate_scatter(loc_v, [idx], ones, mask=inr)
            return 0
        lax.fori_loop(0, E // _LANES, _scan, 0, unroll=8)

        pltpu.sync_copy(loc_v, a_hbm.at[pl.ds(base, _ROWS_PER_W * NP)])

    return _sc_build_adj


_G = 8
_STEPS = B // _G


def _gnn_kernel(A_in, xT, W1, b1, W2, b2, W3, b3,
                Wsn0, bsn0, bn0, g0, bg0,
                Wsn1, bsn1, bn1, g1, bg1,
                Wsn2, bsn2, bn2, g2, bg2,
                Wsn3, bsn3, bn3, g3, bg3,
                Wd1, bd1, Wd2, bd2, w3r, bd3,
                out):
    pid = pl.program_id(0)

    def dot(a, b):
        return jnp.dot(a, b, preferred_element_type=jnp.float32)

    A = A_in[...]
    A_bf = A.astype(jnp.bfloat16)

    def dot_A(m):
        m_hi = m.astype(jnp.bfloat16)
        m_lo = (m - m_hi.astype(jnp.float32)).astype(jnp.bfloat16)
        return (jnp.dot(A_bf, m_hi, preferred_element_type=jnp.float32)
                + jnp.dot(A_bf, m_lo, preferred_element_type=jnp.float32))

    deg = jnp.sum(A, axis=1, keepdims=True)
    invd = 1.0 / jnp.maximum(deg, 1.0)
    dscale = deg * invd

    lanes = lax.broadcasted_iota(jnp.int32, (1, B), 1)
    xTv = xT[...]
    hs = []
    for g in range(_G):
        sel = _f32(lanes == pid * _G + g)
        xc = jnp.sum(xTv * sel, axis=1, keepdims=True)
        hs.append(jax.nn.relu(xc * W1[...] + b1[...]))
    hs = [jax.nn.relu(dot(h, W2[...]) + b2[...]) for h in hs]
    hs = [dot(h, W3[...]) + b3[...] for h in hs]

    layer_refs = ((Wsn0, bsn0, bn0, g0, bg0), (Wsn1, bsn1, bn1, g1, bg1),
                  (Wsn2, bsn2, bn2, g2, bg2), (Wsn3, bsn3, bn3, g3, bg3))
    for Wsn, bsn, bn, g, bg in layer_refs:
        hsms = []
        for p in range(_G // 2):
            h2 = jnp.concatenate([hs[2 * p], hs[2 * p + 1]], axis=0)
            hsm2 = dot(h2, Wsn[...]) + bsn[...]
            hsms += [hsm2[:NP], hsm2[NP:]]
        aggs = []
        for p in range(_G // 2):
            m2 = jnp.concatenate([hsms[2 * p][:, H:],
                                  hsms[2 * p + 1][:, H:]], axis=1)
            agg2 = dot_A(m2)
            aggs += [agg2[:, :H], agg2[:, H:]]
        nb_bias = bn[...] * dscale
        scale = _INV * g[...]
        hs = [jax.nn.relu((hsm[:, :H] + agg * invd + nb_bias + h) * scale
                          + bg[...])
              for hsm, agg, h in zip(hsms, aggs, hs)]

    rowmask = lax.broadcasted_iota(jnp.int32, (NP, H), 0) < N
    hg = jnp.concatenate(
        [jnp.sum(jnp.where(rowmask, h, 0.0), axis=0, keepdims=True) * (1.0 / N)
         for h in hs], axis=0)
    hg = jax.nn.relu(dot(hg, Wd1[...]) + bd1[...])
    hg = jax.nn.relu(dot(hg, Wd2[...]) + bd2[...])
    out[...] = jnp.sum(hg * w3r[...], axis=1, keepdims=True) + bd3[...]


@jax.jit
def _run(xT, edges, flat_weights):
    A = _get_sc_build_adj()(edges).reshape(NP, NP)
    full = lambda shape: pl.BlockSpec(shape, lambda i: (0,) * len(shape))
    in_specs = [full((NP, NP)), full((NP, B))]
    in_specs += [full(w.shape) for w in flat_weights]
    return pl.pallas_call(
        _gnn_kernel,
        grid=(_STEPS,),
        in_specs=in_specs,
        out_specs=pl.BlockSpec((_G, 1), lambda i: (i, 0)),
        out_shape=jax.ShapeDtypeStruct((B, 1), jnp.float32),
    )(A, xT, *flat_weights)


def kernel(x, params, edge_index):
    xT = jnp.zeros((NP, B), jnp.float32).at[:N].set(x.T)

    enc = params["enc"]
    dec = params["dec"]
    flat = [enc[0][0].reshape(1, H), enc[0][1].reshape(1, H),
            enc[1][0], enc[1][1].reshape(1, H),
            enc[2][0], enc[2][1].reshape(1, H)]
    for lp in params["layers"]:
        Wsn = jnp.concatenate([lp["Ws"], lp["Wn"]], axis=1)
        bsn = jnp.concatenate([lp["bs"], jnp.zeros((H,), jnp.float32)]
                              ).reshape(1, 2 * H)
        flat += [Wsn, bsn, lp["bn"].reshape(1, H),
                 lp["g"].reshape(1, H), lp["b"].reshape(1, H)]
    flat += [dec[0][0], dec[0][1].reshape(1, H),
             dec[1][0], dec[1][1].reshape(1, H // 2),
             dec[2][0].reshape(1, H // 2), dec[2][1].reshape(1, 1)]

    return _run(xT, edge_index, tuple(flat))


# speedup vs baseline: 1.3124x; 1.3124x over previous
import functools
import math

import jax
import jax.numpy as jnp
from jax import lax
from jax.experimental import pallas as pl
from jax.experimental.pallas import tpu as pltpu
from jax.experimental.pallas import tpu_sc as plsc

B, N, E, H, NL = 64, 600, 9600, 128, 4
NP = 640
EPS = 1e-5
_INV = 1.0 / math.sqrt(1.0 + EPS)

_NWORKERS = 32
_ROWS_PER_W = NP // _NWORKERS
_LANES = 16


def _f32(x):
    return x.astype(jnp.float32)


@functools.lru_cache(maxsize=None)
def _get_sc_build_adj():
    mesh = plsc.VectorSubcoreMesh(core_axis_name="c", subcore_axis_name="s")

    @functools.partial(
        pl.kernel,
        out_type=jax.ShapeDtypeStruct((NP * NP,), jnp.float32),
        mesh=mesh,
        compiler_params=pltpu.CompilerParams(needs_layout_passes=False),
        scratch_types=[
            pltpu.VMEM((E,), jnp.int32),
            pltpu.VMEM((E,), jnp.int32),
            pltpu.VMEM((_ROWS_PER_W * NP,), jnp.float32),
        ],
    )
    def _sc_build_adj(edges_hbm, a_hbm, src_v, dst_v, loc_v):
        wid = lax.axis_index("s") * 2 + lax.axis_index("c")
        lo = wid * _ROWS_PER_W
        base = lo * NP

        pltpu.sync_copy(edges_hbm.at[0], src_v)
        pltpu.sync_copy(edges_hbm.at[1], dst_v)

        def _zero(i, _):
            loc_v[pl.ds(i * _LANES, _LANES)] = jnp.zeros((_LANES,),
                                                         jnp.float32)
            return 0
        lax.fori_loop(0, _ROWS_PER_W * NP // _LANES, _zero, 0, unroll=8)

        ones = jnp.ones((_LANES,), jnp.float32)

        def _scan(i, _):
            d16 = dst_v[pl.ds(i * _LANES, _LANES)]
            s16 = src_v[pl.ds(i * _LANES, _LANES)]
            inr = (d16 >= lo) & (d16 < lo + _ROWS_PER_W)
            idx = (d16 - lo) * NP + s16
            plsc.addupdate_scatter(loc_v, [idx], ones, mask=inr)
            return 0
        lax.fori_loop(0, E // _LANES, _scan, 0, unroll=8)

        pltpu.sync_copy(loc_v, a_hbm.at[pl.ds(base, _ROWS_PER_W * NP)])

    return _sc_build_adj


_G = 8
_STEPS = B // _G


def _gnn_kernel(A_in, xT, W1, b1, W2, b2, W3, b3,
                Wsn0, bsn0, bn0, g0, bg0,
                Wsn1, bsn1, bn1, g1, bg1,
                Wsn2, bsn2, bn2, g2, bg2,
                Wsn3, bsn3, bn3, g3, bg3,
                Wd1, bd1, Wd2, bd2, w3r, bd3,
                out):
    pid = pl.program_id(0)

    def dot(a, b):
        return jnp.dot(a, b, preferred_element_type=jnp.float32)

    A = A_in[...]
    deg = jnp.sum(A, axis=1, keepdims=True)
    invd = 1.0 / jnp.maximum(deg, 1.0)
    dscale = deg * invd

    lanes = lax.broadcasted_iota(jnp.int32, (1, B), 1)
    xTv = xT[...]
    hs = []
    for g in range(_G):
        sel = _f32(lanes == pid * _G + g)
        xc = jnp.sum(xTv * sel, axis=1, keepdims=True)
        hs.append(jax.nn.relu(xc * W1[...] + b1[...]))
    hs = [jax.nn.relu(dot(h, W2[...]) + b2[...]) for h in hs]
    hs = [dot(h, W3[...]) + b3[...] for h in hs]

    layer_refs = ((Wsn0, bsn0, bn0, g0, bg0), (Wsn1, bsn1, bn1, g1, bg1),
                  (Wsn2, bsn2, bn2, g2, bg2), (Wsn3, bsn3, bn3, g3, bg3))
    for Wsn, bsn, bn, g, bg in layer_refs:
        hsms = []
        for p in range(_G // 2):
            h2 = jnp.concatenate([hs[2 * p], hs[2 * p + 1]], axis=0)
            hsm2 = dot(h2, Wsn[...]) + bsn[...]
            hsms += [hsm2[:NP], hsm2[NP:]]
        aggs = []
        for p in range(_G // 2):
            m2 = jnp.concatenate([hsms[2 * p][:, H:],
                                  hsms[2 * p + 1][:, H:]], axis=1)
            agg2 = dot(A, m2)
            aggs += [agg2[:, :H], agg2[:, H:]]
        nb_bias = bn[...] * dscale
        scale = _INV * g[...]
        hs = [jax.nn.relu((hsm[:, :H] + agg * invd + nb_bias + h) * scale
                          + bg[...])
              for hsm, agg, h in zip(hsms, aggs, hs)]

    rowmask = lax.broadcasted_iota(jnp.int32, (NP, H), 0) < N
    hg = jnp.concatenate(
        [jnp.sum(jnp.where(rowmask, h, 0.0), axis=0, keepdims=True) * (1.0 / N)
         for h in hs], axis=0)
    hg = jax.nn.relu(dot(hg, Wd1[...]) + bd1[...])
    hg = jax.nn.relu(dot(hg, Wd2[...]) + bd2[...])
    out[...] = jnp.sum(hg * w3r[...], axis=1, keepdims=True) + bd3[...]


@jax.jit
def _run(xT, edges, flat_weights):
    A = _get_sc_build_adj()(edges).reshape(NP, NP)
    full = lambda shape: pl.BlockSpec(shape, lambda i: (0,) * len(shape))
    in_specs = [full((NP, NP)), full((NP, B))]
    in_specs += [full(w.shape) for w in flat_weights]
    return pl.pallas_call(
        _gnn_kernel,
        grid=(_STEPS,),
        in_specs=in_specs,
        out_specs=pl.BlockSpec((_G, 1), lambda i: (i, 0)),
        out_shape=jax.ShapeDtypeStruct((B, 1), jnp.float32),
    )(A, xT, *flat_weights)


def kernel(x, params, edge_index):
    xT = jnp.zeros((NP, B), jnp.float32).at[:N].set(x.T)

    enc = params["enc"]
    dec = params["dec"]
    flat = [enc[0][0].reshape(1, H), enc[0][1].reshape(1, H),
            enc[1][0], enc[1][1].reshape(1, H),
            enc[2][0], enc[2][1].reshape(1, H)]
    for lp in params["layers"]:
        Wsn = jnp.concatenate([lp["Ws"], lp["Wn"]], axis=1)
        bsn = jnp.concatenate([lp["bs"], jnp.zeros((H,), jnp.float32)]
                              ).reshape(1, 2 * H)
        flat += [Wsn, bsn, lp["bn"].reshape(1, H),
                 lp["g"].reshape(1, H), lp["b"].reshape(1, H)]
    flat += [dec[0][0], dec[0][1].reshape(1, H),
             dec[1][0], dec[1][1].reshape(1, H // 2),
             dec[2][0].reshape(1, H // 2), dec[2][1].reshape(1, 1)]

    return _run(xT, edge_index, tuple(flat))


# speedup vs baseline: 1.3425x; 1.0229x over previous
import functools
import math

import jax
import jax.numpy as jnp
from jax import lax
from jax.experimental import pallas as pl
from jax.experimental.pallas import tpu as pltpu
from jax.experimental.pallas import tpu_sc as plsc

B, N, E, H, NL = 64, 600, 9600, 128, 4
NP = 640
EPS = 1e-5
_INV = 1.0 / math.sqrt(1.0 + EPS)

_NWORKERS = 32
_ROWS_PER_W = NP // _NWORKERS
_LANES = 16


def _f32(x):
    return x.astype(jnp.float32)


@functools.lru_cache(maxsize=None)
def _get_sc_build_adj():
    mesh = plsc.VectorSubcoreMesh(core_axis_name="c", subcore_axis_name="s")

    @functools.partial(
        pl.kernel,
        out_type=jax.ShapeDtypeStruct((NP * NP,), jnp.float32),
        mesh=mesh,
        compiler_params=pltpu.CompilerParams(needs_layout_passes=False),
        scratch_types=[
            pltpu.VMEM((E,), jnp.int32),
            pltpu.VMEM((E,), jnp.int32),
            pltpu.VMEM((_ROWS_PER_W * NP,), jnp.float32),
        ],
    )
    def _sc_build_adj(edges_hbm, a_hbm, src_v, dst_v, loc_v):
        wid = lax.axis_index("s") * 2 + lax.axis_index("c")
        lo = wid * _ROWS_PER_W
        base = lo * NP

        pltpu.sync_copy(edges_hbm.at[0], src_v)
        pltpu.sync_copy(edges_hbm.at[1], dst_v)

        def _zero(i, _):
            loc_v[pl.ds(i * _LANES, _LANES)] = jnp.zeros((_LANES,),
                                                         jnp.float32)
            return 0
        lax.fori_loop(0, _ROWS_PER_W * NP // _LANES, _zero, 0, unroll=8)

        ones = jnp.ones((_LANES,), jnp.float32)

        def _scan(i, _):
            d16 = dst_v[pl.ds(i * _LANES, _LANES)]
            s16 = src_v[pl.ds(i * _LANES, _LANES)]
            inr = (d16 >= lo) & (d16 < lo + _ROWS_PER_W)
            idx = (d16 - lo) * NP + s16
            plsc.addupdate_scatter(loc_v, [idx], ones, mask=inr)
            return 0
        lax.fori_loop(0, E // _LANES, _scan, 0, unroll=8)

        pltpu.sync_copy(loc_v, a_hbm.at[pl.ds(base, _ROWS_PER_W * NP)])

    return _sc_build_adj


_G = 8
_STEPS = B // _G


def _gnn_kernel(A_in, x_in, W1, b1, W2, b2, W3, b3,
                Ws0, Wn0, bs0, bn0, g0, bg0,
                Ws1, Wn1, bs1, bn1, g1, bg1,
                Ws2, Wn2, bs2, bn2, g2, bg2,
                Ws3, Wn3, bs3, bn3, g3, bg3,
                Wd1, bd1, Wd2, bd2, w3r, bd3,
                out):
    pid = pl.program_id(0)

    def dot(a, b):
        return jnp.dot(a, b, preferred_element_type=jnp.float32)

    A = A_in[...]
    deg = jnp.sum(A, axis=1, keepdims=True)
    invd = 1.0 / jnp.maximum(deg, 1.0)
    dscale = deg * invd

    gsel = _f32(lax.broadcasted_iota(jnp.int32, (_G, B), 1)
                == _G * pid + lax.broadcasted_iota(jnp.int32, (_G, B), 0))
    xcols = lax.dot_general(x_in[...], gsel, (((0,), (1,)), ((), ())),
                            preferred_element_type=jnp.float32)
    xcols = jnp.concatenate(
        [xcols, jnp.zeros((NP - N, _G), jnp.float32)], axis=0)
    w1cat = jnp.concatenate([W1[...]] * _G, axis=1)
    gmask = (lax.broadcasted_iota(jnp.int32, (_G, _G * H), 0)
             == lax.broadcasted_iota(jnp.int32, (_G, _G * H), 1) // H)
    W1blk = jnp.where(gmask, w1cat, 0.0)
    b1cat = jnp.concatenate([b1[...]] * _G, axis=1)
    h0all = jax.nn.relu(dot(xcols, W1blk) + b1cat)
    hs = [h0all[:, g * H:(g + 1) * H] for g in range(_G)]
    hs = [jax.nn.relu(dot(h, W2[...]) + b2[...]) for h in hs]
    hs = [dot(h, W3[...]) + b3[...] for h in hs]

    layer_refs = ((Ws0, Wn0, bs0, bn0, g0, bg0), (Ws1, Wn1, bs1, bn1, g1, bg1),
                  (Ws2, Wn2, bs2, bn2, g2, bg2), (Ws3, Wn3, bs3, bn3, g3, bg3))
    for Ws, Wn, bs, bn, g, bg in layer_refs:
        Wsn = jnp.concatenate([Ws[...], Wn[...]], axis=1)
        hsms = []
        for p in range(_G // 2):
            h2 = jnp.concatenate([hs[2 * p], hs[2 * p + 1]], axis=0)
            hsm2 = dot(h2, Wsn)
            hsms += [hsm2[:NP], hsm2[NP:]]
        aggs = []
        for p in range(_G // 2):
            m2 = jnp.concatenate([hsms[2 * p][:, H:],
                                  hsms[2 * p + 1][:, H:]], axis=1)
            agg2 = dot(A, m2)
            aggs += [agg2[:, :H], agg2[:, H:]]
        nb_bias = bs[...] + bn[...] * dscale
        scale = _INV * g[...]
        hs = [jax.nn.relu((hsm[:, :H] + agg * invd + nb_bias + h) * scale
                          + bg[...])
              for hsm, agg, h in zip(hsms, aggs, hs)]

    rowmask = lax.broadcasted_iota(jnp.int32, (NP, H), 0) < N
    hg = jnp.concatenate(
        [jnp.sum(jnp.where(rowmask, h, 0.0), axis=0, keepdims=True) * (1.0 / N)
         for h in hs], axis=0)
    hg = jax.nn.relu(dot(hg, Wd1[...]) + bd1[...])
    hg = jax.nn.relu(dot(hg, Wd2[...]) + bd2[...])
    out[...] = jnp.sum(hg * w3r[...], axis=1, keepdims=True) + bd3[...]


@jax.jit
def _run(x, edges, flat_weights):
    A = _get_sc_build_adj()(edges).reshape(NP, NP)
    full = lambda shape: pl.BlockSpec(shape, lambda i: (0,) * len(shape))
    in_specs = [full((NP, NP)), full((B, N))]
    in_specs += [full(w.shape) for w in flat_weights]
    return pl.pallas_call(
        _gnn_kernel,
        grid=(_STEPS,),
        in_specs=in_specs,
        out_specs=pl.BlockSpec((_G, 1), lambda i: (i, 0)),
        out_shape=jax.ShapeDtypeStruct((B, 1), jnp.float32),
    )(A, x, *flat_weights)


def kernel(x, params, edge_index):
    enc = params["enc"]
    dec = params["dec"]
    flat = [enc[0][0].reshape(1, H), enc[0][1].reshape(1, H),
            enc[1][0], enc[1][1].reshape(1, H),
            enc[2][0], enc[2][1].reshape(1, H)]
    for lp in params["layers"]:
        flat += [lp["Ws"], lp["Wn"], lp["bs"].reshape(1, H),
                 lp["bn"].reshape(1, H),
                 lp["g"].reshape(1, H), lp["b"].reshape(1, H)]
    flat += [dec[0][0], dec[0][1].reshape(1, H),
             dec[1][0], dec[1][1].reshape(1, H // 2),
             dec[2][0].reshape(1, H // 2), dec[2][1].reshape(1, 1)]

    return _run(x, edge_index, tuple(flat))


# speedup vs baseline: 1.3996x; 1.0425x over previous
import functools
import math

import jax
import jax.numpy as jnp
from jax import lax
from jax.experimental import pallas as pl
from jax.experimental.pallas import tpu as pltpu
from jax.experimental.pallas import tpu_sc as plsc

B, N, E, H, NL = 64, 600, 9600, 128, 4
NP = 640
EPS = 1e-5
_INV = 1.0 / math.sqrt(1.0 + EPS)

_NWORKERS = 32
_ROWS_PER_W = NP // _NWORKERS
_LANES = 16


def _f32(x):
    return x.astype(jnp.float32)


@functools.lru_cache(maxsize=None)
def _get_sc_build_adj():
    mesh = plsc.VectorSubcoreMesh(core_axis_name="c", subcore_axis_name="s")

    @functools.partial(
        pl.kernel,
        out_type=jax.ShapeDtypeStruct((NP * NP,), jnp.float32),
        mesh=mesh,
        compiler_params=pltpu.CompilerParams(needs_layout_passes=False),
        scratch_types=[
            pltpu.VMEM((E,), jnp.int32),
            pltpu.VMEM((E,), jnp.int32),
            pltpu.VMEM((_ROWS_PER_W * NP,), jnp.float32),
        ],
    )
    def _sc_build_adj(edges_hbm, a_hbm, src_v, dst_v, loc_v):
        wid = lax.axis_index("s") * 2 + lax.axis_index("c")
        lo = wid * _ROWS_PER_W
        base = lo * NP

        pltpu.sync_copy(edges_hbm.at[0], src_v)
        pltpu.sync_copy(edges_hbm.at[1], dst_v)

        def _zero(i, _):
            loc_v[pl.ds(i * _LANES, _LANES)] = jnp.zeros((_LANES,),
                                                         jnp.float32)
            return 0
        lax.fori_loop(0, _ROWS_PER_W * NP // _LANES, _zero, 0, unroll=8)

        ones = jnp.ones((_LANES,), jnp.float32)

        def _scan(i, _):
            d16 = dst_v[pl.ds(i * _LANES, _LANES)]
            s16 = src_v[pl.ds(i * _LANES, _LANES)]
            inr = (d16 >= lo) & (d16 < lo + _ROWS_PER_W)
            idx = (d16 - lo) * NP + s16
            plsc.addupdate_scatter(loc_v, [idx], ones, mask=inr)
            return 0
        lax.fori_loop(0, E // _LANES, _scan, 0, unroll=8)

        pltpu.sync_copy(loc_v, a_hbm.at[pl.ds(base, _ROWS_PER_W * NP)])

    return _sc_build_adj


_G = 16
_STEPS = B // _G


def _gnn_kernel(A_in, x_in, W1, b1, W2, b2, W3, b3,
                Ws0, Wn0, bs0, bn0, g0, bg0,
                Ws1, Wn1, bs1, bn1, g1, bg1,
                Ws2, Wn2, bs2, bn2, g2, bg2,
                Ws3, Wn3, bs3, bn3, g3, bg3,
                Wd1, bd1, Wd2, bd2, w3r, bd3,
                out):
    pid = pl.program_id(0)

    def dot(a, b):
        return jnp.dot(a, b, preferred_element_type=jnp.float32)

    A = A_in[...]
    deg = jnp.sum(A, axis=1, keepdims=True)
    invd = 1.0 / jnp.maximum(deg, 1.0)
    dscale = deg * invd

    gsel = _f32(lax.broadcasted_iota(jnp.int32, (_G, B), 1)
                == _G * pid + lax.broadcasted_iota(jnp.int32, (_G, B), 0))
    xcols = lax.dot_general(x_in[...], gsel, (((0,), (1,)), ((), ())),
                            preferred_element_type=jnp.float32)
    xcols = jnp.concatenate(
        [xcols, jnp.zeros((NP - N, _G), jnp.float32)], axis=0)
    w1cat = jnp.concatenate([W1[...]] * _G, axis=1)
    gmask = (lax.broadcasted_iota(jnp.int32, (_G, _G * H), 0)
             == lax.broadcasted_iota(jnp.int32, (_G, _G * H), 1) // H)
    W1blk = jnp.where(gmask, w1cat, 0.0)
    b1cat = jnp.concatenate([b1[...]] * _G, axis=1)
    h0all = jax.nn.relu(dot(xcols, W1blk) + b1cat)
    hs = [h0all[:, g * H:(g + 1) * H] for g in range(_G)]
    hs = [jax.nn.relu(dot(h, W2[...]) + b2[...]) for h in hs]
    hs = [dot(h, W3[...]) + b3[...] for h in hs]

    layer_refs = ((Ws0, Wn0, bs0, bn0, g0, bg0), (Ws1, Wn1, bs1, bn1, g1, bg1),
                  (Ws2, Wn2, bs2, bn2, g2, bg2), (Ws3, Wn3, bs3, bn3, g3, bg3))
    for Ws, Wn, bs, bn, g, bg in layer_refs:
        Wsn = jnp.concatenate([Ws[...], Wn[...]], axis=1)
        hsms = []
        for p in range(_G // 2):
            h2 = jnp.concatenate([hs[2 * p], hs[2 * p + 1]], axis=0)
            hsm2 = dot(h2, Wsn)
            hsms += [hsm2[:NP], hsm2[NP:]]
        aggs = []
        for p in range(_G // 2):
            m2 = jnp.concatenate([hsms[2 * p][:, H:],
                                  hsms[2 * p + 1][:, H:]], axis=1)
            agg2 = dot(A, m2)
            aggs += [agg2[:, :H], agg2[:, H:]]
        nb_bias = bs[...] + bn[...] * dscale
        scale = _INV * g[...]
        hs = [jax.nn.relu((hsm[:, :H] + agg * invd + nb_bias + h) * scale
                          + bg[...])
              for hsm, agg, h in zip(hsms, aggs, hs)]

    rowmask = lax.broadcasted_iota(jnp.int32, (NP, H), 0) < N
    hg = jnp.concatenate(
        [jnp.sum(jnp.where(rowmask, h, 0.0), axis=0, keepdims=True) * (1.0 / N)
         for h in hs], axis=0)
    hg = jax.nn.relu(dot(hg, Wd1[...]) + bd1[...])
    hg = jax.nn.relu(dot(hg, Wd2[...]) + bd2[...])
    out[...] = jnp.sum(hg * w3r[...], axis=1, keepdims=True) + bd3[...]


@jax.jit
def _run(x, edges, flat_weights):
    A = _get_sc_build_adj()(edges).reshape(NP, NP)
    full = lambda shape: pl.BlockSpec(shape, lambda i: (0,) * len(shape))
    in_specs = [full((NP, NP)), full((B, N))]
    in_specs += [full(w.shape) for w in flat_weights]
    return pl.pallas_call(
        _gnn_kernel,
        grid=(_STEPS,),
        in_specs=in_specs,
        out_specs=pl.BlockSpec((_G, 1), lambda i: (i, 0)),
        out_shape=jax.ShapeDtypeStruct((B, 1), jnp.float32),
    )(A, x, *flat_weights)


def kernel(x, params, edge_index):
    enc = params["enc"]
    dec = params["dec"]
    flat = [enc[0][0].reshape(1, H), enc[0][1].reshape(1, H),
            enc[1][0], enc[1][1].reshape(1, H),
            enc[2][0], enc[2][1].reshape(1, H)]
    for lp in params["layers"]:
        flat += [lp["Ws"], lp["Wn"], lp["bs"].reshape(1, H),
                 lp["bn"].reshape(1, H),
                 lp["g"].reshape(1, H), lp["b"].reshape(1, H)]
    flat += [dec[0][0], dec[0][1].reshape(1, H),
             dec[1][0], dec[1][1].reshape(1, H // 2),
             dec[2][0].reshape(1, H // 2), dec[2][1].reshape(1, 1)]

    return _run(x, edge_index, tuple(flat))


# speedup vs baseline: 1.4072x; 1.0055x over previous
import functools
import math

import jax
import jax.numpy as jnp
from jax import lax
from jax.experimental import pallas as pl
from jax.experimental.pallas import tpu as pltpu
from jax.experimental.pallas import tpu_sc as plsc

B, N, E, H, NL = 64, 600, 9600, 128, 4
NP = 640
EPS = 1e-5
_INV = 1.0 / math.sqrt(1.0 + EPS)

_NWORKERS = 32
_ROWS_PER_W = NP // _NWORKERS
_LANES = 16


def _f32(x):
    return x.astype(jnp.float32)


@functools.lru_cache(maxsize=None)
def _get_sc_build_adj():
    mesh = plsc.VectorSubcoreMesh(core_axis_name="c", subcore_axis_name="s")

    @functools.partial(
        pl.kernel,
        out_type=jax.ShapeDtypeStruct((NP * NP,), jnp.float32),
        mesh=mesh,
        compiler_params=pltpu.CompilerParams(needs_layout_passes=False),
        scratch_types=[
            pltpu.VMEM((E,), jnp.int32),
            pltpu.VMEM((E,), jnp.int32),
            pltpu.VMEM((_ROWS_PER_W * NP,), jnp.float32),
            pltpu.SemaphoreType.DMA,
            pltpu.SemaphoreType.DMA,
        ],
    )
    def _sc_build_adj(edges_hbm, a_hbm, src_v, dst_v, loc_v, sem_s, sem_d):
        wid = lax.axis_index("s") * 2 + lax.axis_index("c")
        lo = wid * _ROWS_PER_W
        base = lo * NP

        cp_s = pltpu.async_copy(edges_hbm.at[0], src_v, sem_s)
        cp_d = pltpu.async_copy(edges_hbm.at[1], dst_v, sem_d)

        def _zero(i, _):
            loc_v[pl.ds(i * _LANES, _LANES)] = jnp.zeros((_LANES,),
                                                         jnp.float32)
            return 0
        lax.fori_loop(0, _ROWS_PER_W * NP // _LANES, _zero, 0, unroll=8)

        cp_s.wait()
        cp_d.wait()
        ones = jnp.ones((_LANES,), jnp.float32)

        def _scan(i, _):
            d16 = dst_v[pl.ds(i * _LANES, _LANES)]
            s16 = src_v[pl.ds(i * _LANES, _LANES)]
            inr = (d16 - lo).astype(jnp.uint32) < _ROWS_PER_W
            idx = (d16 - lo) * NP + s16
            plsc.addupdate_scatter(loc_v, [idx], ones, mask=inr)
            return 0
        lax.fori_loop(0, E // _LANES, _scan, 0, unroll=8)

        pltpu.sync_copy(loc_v, a_hbm.at[pl.ds(base, _ROWS_PER_W * NP)])

    return _sc_build_adj


_G = 16
_STEPS = B // _G


def _gnn_kernel(A_in, x_in, W1, b1, W2, b2, W3, b3,
                Ws0, Wn0, bs0, bn0, g0, bg0,
                Ws1, Wn1, bs1, bn1, g1, bg1,
                Ws2, Wn2, bs2, bn2, g2, bg2,
                Ws3, Wn3, bs3, bn3, g3, bg3,
                Wd1, bd1, Wd2, bd2, w3r, bd3,
                out):
    pid = pl.program_id(0)

    def dot(a, b):
        return jnp.dot(a, b, preferred_element_type=jnp.float32)

    A = A_in[...]
    deg = jnp.sum(A, axis=1, keepdims=True)
    invd = 1.0 / jnp.maximum(deg, 1.0)
    dscale = deg * invd

    gsel = _f32(lax.broadcasted_iota(jnp.int32, (_G, B), 1)
                == _G * pid + lax.broadcasted_iota(jnp.int32, (_G, B), 0))
    xcols = lax.dot_general(x_in[...], gsel, (((0,), (1,)), ((), ())),
                            preferred_element_type=jnp.float32)
    xcols = jnp.concatenate(
        [xcols, jnp.zeros((NP - N, _G), jnp.float32)], axis=0)
    w1cat = jnp.concatenate([W1[...]] * _G, axis=1)
    gmask = (lax.broadcasted_iota(jnp.int32, (_G, _G * H), 0)
             == lax.broadcasted_iota(jnp.int32, (_G, _G * H), 1) // H)
    W1blk = jnp.where(gmask, w1cat, 0.0)
    b1cat = jnp.concatenate([b1[...]] * _G, axis=1)
    h0all = jax.nn.relu(dot(xcols, W1blk) + b1cat)
    hs = [h0all[:, g * H:(g + 1) * H] for g in range(_G)]
    hs = [jax.nn.relu(dot(h, W2[...]) + b2[...]) for h in hs]
    hs = [dot(h, W3[...]) + b3[...] for h in hs]

    layer_refs = ((Ws0, Wn0, bs0, bn0, g0, bg0), (Ws1, Wn1, bs1, bn1, g1, bg1),
                  (Ws2, Wn2, bs2, bn2, g2, bg2), (Ws3, Wn3, bs3, bn3, g3, bg3))
    for Ws, Wn, bs, bn, g, bg in layer_refs:
        Wsn = jnp.concatenate([Ws[...], Wn[...]], axis=1)
        hsms = []
        for p in range(_G // 2):
            h2 = jnp.concatenate([hs[2 * p], hs[2 * p + 1]], axis=0)
            hsm2 = dot(h2, Wsn)
            hsms += [hsm2[:NP], hsm2[NP:]]
        aggs = []
        for p in range(_G // 2):
            m2 = jnp.concatenate([hsms[2 * p][:, H:],
                                  hsms[2 * p + 1][:, H:]], axis=1)
            agg2 = dot(A, m2)
            aggs += [agg2[:, :H], agg2[:, H:]]
        nb_bias = bs[...] + bn[...] * dscale
        scale = _INV * g[...]
        hs = [jax.nn.relu((hsm[:, :H] + agg * invd + nb_bias + h) * scale
                          + bg[...])
              for hsm, agg, h in zip(hsms, aggs, hs)]

    rowmask = lax.broadcasted_iota(jnp.int32, (NP, H), 0) < N
    hg = jnp.concatenate(
        [jnp.sum(jnp.where(rowmask, h, 0.0), axis=0, keepdims=True) * (1.0 / N)
         for h in hs], axis=0)
    hg = jax.nn.relu(dot(hg, Wd1[...]) + bd1[...])
    hg = jax.nn.relu(dot(hg, Wd2[...]) + bd2[...])
    out[...] = jnp.sum(hg * w3r[...], axis=1, keepdims=True) + bd3[...]


@jax.jit
def _run(x, edges, flat_weights):
    A = _get_sc_build_adj()(edges).reshape(NP, NP)
    full = lambda shape: pl.BlockSpec(shape, lambda i: (0,) * len(shape))
    in_specs = [full((NP, NP)), full((B, N))]
    in_specs += [full(w.shape) for w in flat_weights]
    return pl.pallas_call(
        _gnn_kernel,
        grid=(_STEPS,),
        in_specs=in_specs,
        out_specs=pl.BlockSpec((_G, 1), lambda i: (i, 0)),
        out_shape=jax.ShapeDtypeStruct((B, 1), jnp.float32),
    )(A, x, *flat_weights)


def kernel(x, params, edge_index):
    enc = params["enc"]
    dec = params["dec"]
    flat = [enc[0][0].reshape(1, H), enc[0][1].reshape(1, H),
            enc[1][0], enc[1][1].reshape(1, H),
            enc[2][0], enc[2][1].reshape(1, H)]
    for lp in params["layers"]:
        flat += [lp["Ws"], lp["Wn"], lp["bs"].reshape(1, H),
                 lp["bn"].reshape(1, H),
                 lp["g"].reshape(1, H), lp["b"].reshape(1, H)]
    flat += [dec[0][0], dec[0][1].reshape(1, H),
             dec[1][0], dec[1][1].reshape(1, H // 2),
             dec[2][0].reshape(1, H // 2), dec[2][1].reshape(1, 1)]

    return _run(x, edge_index, tuple(flat))
